# baseline (speedup 1.0000x reference)
import functools

import jax
import jax.numpy as jnp
from jax import lax
from jax.experimental import pallas as pl
from jax.experimental.pallas import tpu as pltpu
from jax.experimental.pallas import tpu_sc as plsc

_NC = 2
_NS = 16
_ROWS_PER_W = 8


def _sc_table_kernel(pos_hbm, kf_hbm, idx_hbm, out_hbm, idx_v, row_v, kf_v,
                     sem, *, n_workers):
    wid = lax.axis_index("s") * _NC + lax.axis_index("c")

    @pl.when(wid < n_workers)
    def _():
        base = wid * _ROWS_PER_W
        pltpu.sync_copy(idx_hbm.at[pl.ds(base, _ROWS_PER_W)], idx_v)
        pltpu.sync_copy(pos_hbm.at[pl.ds(base, _ROWS_PER_W)], row_v)
        pltpu.async_copy(kf_hbm.at[idx_v], kf_v, sem).wait()
        d = row_v.shape[1]
        for r in range(_ROWS_PER_W):
            for c in range(d // 16):
                sl = (r, pl.ds(c * 16, 16))
                row_v[sl] = row_v[sl] + kf_v[sl]
        pltpu.sync_copy(row_v, out_hbm.at[pl.ds(base, _ROWS_PER_W)])


def _build_table_sc(pos_embed, kf_idx, kf_embed):
    s_len, d = pos_embed.shape
    n_workers = s_len // _ROWS_PER_W
    mesh = plsc.VectorSubcoreMesh(core_axis_name="c", subcore_axis_name="s")
    return pl.kernel(
        functools.partial(_sc_table_kernel, n_workers=n_workers),
        out_type=jax.ShapeDtypeStruct((s_len, d), jnp.float32),
        mesh=mesh,
        scratch_types=[
            pltpu.VMEM((_ROWS_PER_W,), jnp.int32),
            pltpu.VMEM((_ROWS_PER_W, d), jnp.float32),
            pltpu.VMEM((_ROWS_PER_W, d), jnp.float32),
            pltpu.SemaphoreType.DMA,
        ],
    )(pos_embed, kf_embed, kf_idx)


def _tc_ln_kernel(tab_ref, w_ref, b_ref, x_ref, o_ref):
    bblk, s_len, d = x_ref.shape
    jmat = jnp.full((d, d), 1.0 / d, dtype=jnp.float32)
    emb = (x_ref[...] + tab_ref[...][None, :, :]).reshape(bblk * s_len, d)
    mean = lax.dot(emb, jmat,
                   preferred_element_type=jnp.float32)
    diff = emb - mean
    s2 = jnp.sum(diff * diff, axis=-1, keepdims=True)
    scale = lax.rsqrt(s2 + jnp.float32(d * 1e-5))
    o_ref[...] = (diff * (scale * w_ref[...]) + b_ref[...]).reshape(
        bblk, s_len, d)


def kernel(x, pos_embed, kf_embed, ln_weight, ln_bias, n_past, n_future,
           n_trans):
    b, s_len, d = x.shape
    bblk = 64

    pos_index = jnp.arange(s_len, dtype=jnp.int32)
    n_past = jnp.asarray(n_past, jnp.int32)
    n_trans = jnp.asarray(n_trans, jnp.int32)
    n_position = n_past + jnp.asarray(n_future, jnp.int32) + n_trans
    kf_idx = jnp.where((pos_index >= n_past) & (pos_index < n_past + n_trans),
                       jnp.int32(1), jnp.int32(0))
    kf_idx = jnp.where(pos_index >= n_position, jnp.int32(2), kf_idx)

    table = _build_table_sc(pos_embed, kf_idx, kf_embed)

    return pl.pallas_call(
        _tc_ln_kernel,
        grid=(b // bblk,),
        in_specs=[
            pl.BlockSpec((s_len, d), lambda i: (0, 0)),
            pl.BlockSpec((1, d), lambda i: (0, 0)),
            pl.BlockSpec((1, d), lambda i: (0, 0)),
            pl.BlockSpec((bblk, s_len, d), lambda i: (i, 0, 0)),
        ],
        out_specs=pl.BlockSpec((bblk, s_len, d), lambda i: (i, 0, 0)),
        out_shape=jax.ShapeDtypeStruct((b, s_len, d), x.dtype),
        compiler_params=pltpu.CompilerParams(
            dimension_semantics=("arbitrary",)),
    )(table, (ln_weight * jnp.sqrt(jnp.float32(d))).reshape(1, d),
      ln_bias.reshape(1, d), x)


# speedup vs baseline: 1.0107x; 1.0107x over previous
import functools

import jax
import jax.numpy as jnp
from jax import lax
from jax.experimental import pallas as pl
from jax.experimental.pallas import tpu as pltpu
from jax.experimental.pallas import tpu_sc as plsc

_NC = 2
_NS = 16
_ROWS_PER_W = 8


def _sc_gather_kernel(kf_hbm, idx_hbm, out_hbm, idx_v, rows_v, sem,
                      *, n_workers):
    wid = lax.axis_index("s") * _NC + lax.axis_index("c")

    @pl.when(wid < n_workers)
    def _():
        base = wid * _ROWS_PER_W
        pltpu.sync_copy(idx_hbm.at[pl.ds(base, _ROWS_PER_W)], idx_v)
        pltpu.async_copy(kf_hbm.at[idx_v], rows_v, sem).wait()
        pltpu.sync_copy(rows_v, out_hbm.at[pl.ds(base, _ROWS_PER_W)])


def _gather_kf_sc(kf_embed, kf_idx, d):
    s_len = kf_idx.shape[0]
    n_workers = s_len // _ROWS_PER_W
    mesh = plsc.VectorSubcoreMesh(core_axis_name="c", subcore_axis_name="s")
    return pl.kernel(
        functools.partial(_sc_gather_kernel, n_workers=n_workers),
        out_type=jax.ShapeDtypeStruct((s_len, d), jnp.float32),
        mesh=mesh,
        scratch_types=[
            pltpu.VMEM((_ROWS_PER_W,), jnp.int32),
            pltpu.VMEM((_ROWS_PER_W, d), jnp.float32),
            pltpu.SemaphoreType.DMA,
        ],
    )(kf_embed, kf_idx)


def _tc_ln_kernel(pos_ref, kf_ref, w_ref, b_ref, x_ref, o_ref):
    bblk, s_len, d = x_ref.shape
    jmat = jnp.full((d, d), 1.0 / d, dtype=jnp.float32)
    add = (pos_ref[...] + kf_ref[...])[None, :, :]
    emb = (x_ref[...] + add).reshape(bblk * s_len, d)
    mean = lax.dot(emb, jmat,
                   preferred_element_type=jnp.float32)
    diff = emb - mean
    s2 = jnp.sum(diff * diff, axis=-1, keepdims=True)
    w = w_ref[...] * jnp.float32(d ** 0.5)
    scale = lax.rsqrt(s2 + jnp.float32(d * 1e-5))
    o_ref[...] = (diff * (scale * w) + b_ref[...]).reshape(bblk, s_len, d)


def kernel(x, pos_embed, kf_embed, ln_weight, ln_bias, n_past, n_future,
           n_trans):
    b, s_len, d = x.shape
    bblk = 64

    pos_index = jnp.arange(s_len, dtype=jnp.int32)
    n_past = jnp.asarray(n_past, jnp.int32)
    n_trans = jnp.asarray(n_trans, jnp.int32)
    n_position = n_past + jnp.asarray(n_future, jnp.int32) + n_trans
    kf_idx = jnp.where((pos_index >= n_past) & (pos_index < n_past + n_trans),
                       jnp.int32(1), jnp.int32(0))
    kf_idx = jnp.where(pos_index >= n_position, jnp.int32(2), kf_idx)

    kf_table = _gather_kf_sc(kf_embed, kf_idx, d)

    return pl.pallas_call(
        _tc_ln_kernel,
        grid=(b // bblk,),
        in_specs=[
            pl.BlockSpec((s_len, d), lambda i: (0, 0)),
            pl.BlockSpec((s_len, d), lambda i: (0, 0)),
            pl.BlockSpec((1, d), lambda i: (0, 0)),
            pl.BlockSpec((1, d), lambda i: (0, 0)),
            pl.BlockSpec((bblk, s_len, d), lambda i: (i, 0, 0)),
        ],
        out_specs=pl.BlockSpec((bblk, s_len, d), lambda i: (i, 0, 0)),
        out_shape=jax.ShapeDtypeStruct((b, s_len, d), x.dtype),
        compiler_params=pltpu.CompilerParams(
            dimension_semantics=("arbitrary",)),
    )(pos_embed, kf_table, ln_weight.reshape(1, d), ln_bias.reshape(1, d), x)


# speedup vs baseline: 1.0213x; 1.0105x over previous
import functools

import jax
import jax.numpy as jnp
from jax import lax
from jax.experimental import pallas as pl
from jax.experimental.pallas import tpu as pltpu
from jax.experimental.pallas import tpu_sc as plsc

_NC = 2
_NS = 16
_ROWS_PER_W = 8


def _sc_gather_kernel(kf_hbm, idx_hbm, out_hbm, idx16, rows16, idx8, rows8,
                      sem, *, n_full, tail_base):
    wid = lax.axis_index("s") + lax.axis_index("c")

    @pl.when(wid < n_full)
    def _():
        base = wid * 16
        pltpu.sync_copy(idx_hbm.at[pl.ds(base, 16)], idx16)
        pltpu.async_copy(kf_hbm.at[idx16], rows16, sem).wait()
        pltpu.sync_copy(rows16, out_hbm.at[pl.ds(base, 16)])

    @pl.when(wid == n_full)
    def _():
        pltpu.sync_copy(idx_hbm.at[pl.ds(tail_base, 8)], idx8)
        pltpu.async_copy(kf_hbm.at[idx8], rows8, sem).wait()
        pltpu.sync_copy(rows8, out_hbm.at[pl.ds(tail_base, 8)])


def _gather_kf_sc(kf_embed, kf_idx, d):
    s_len = kf_idx.shape[0]
    n_full = s_len // 16
    mesh = plsc.VectorSubcoreMesh(core_axis_name="c", subcore_axis_name="s",
                                  num_cores=1)
    return pl.kernel(
        functools.partial(_sc_gather_kernel, n_full=n_full,
                          tail_base=n_full * 16),
        out_type=jax.ShapeDtypeStruct((s_len, d), jnp.float32),
        mesh=mesh,
        scratch_types=[
            pltpu.VMEM((16,), jnp.int32),
            pltpu.VMEM((16, d), jnp.float32),
            pltpu.VMEM((8,), jnp.int32),
            pltpu.VMEM((8, d), jnp.float32),
            pltpu.SemaphoreType.DMA,
        ],
    )(kf_embed, kf_idx)


def _tc_ln_kernel(pos_ref, kf_ref, w_ref, b_ref, x_ref, o_ref):
    bblk, s_len, d = x_ref.shape
    jmat = jnp.full((d, d), 1.0 / d, dtype=jnp.float32)
    add = (pos_ref[...] + kf_ref[...])[None, :, :]
    emb = (x_ref[...] + add).reshape(bblk * s_len, d)
    mean = lax.dot(emb, jmat,
                   preferred_element_type=jnp.float32)
    diff = emb - mean
    s2 = jnp.sum(diff * diff, axis=-1, keepdims=True)
    w = w_ref[...] * jnp.float32(d ** 0.5)
    scale = lax.rsqrt(s2 + jnp.float32(d * 1e-5))
    o_ref[...] = (diff * (scale * w) + b_ref[...]).reshape(bblk, s_len, d)


def kernel(x, pos_embed, kf_embed, ln_weight, ln_bias, n_past, n_future,
           n_trans):
    b, s_len, d = x.shape
    bblk = 64

    pos_index = jnp.arange(s_len, dtype=jnp.int32)
    n_past = jnp.asarray(n_past, jnp.int32)
    n_trans = jnp.asarray(n_trans, jnp.int32)
    n_position = n_past + jnp.asarray(n_future, jnp.int32) + n_trans
    kf_idx = jnp.where((pos_index >= n_past) & (pos_index < n_past + n_trans),
                       jnp.int32(1), jnp.int32(0))
    kf_idx = jnp.where(pos_index >= n_position, jnp.int32(2), kf_idx)

    kf_table = _gather_kf_sc(kf_embed, kf_idx, d)

    return pl.pallas_call(
        _tc_ln_kernel,
        grid=(b // bblk,),
        in_specs=[
            pl.BlockSpec((s_len, d), lambda i: (0, 0)),
            pl.BlockSpec((s_len, d), lambda i: (0, 0)),
            pl.BlockSpec((1, d), lambda i: (0, 0)),
            pl.BlockSpec((1, d), lambda i: (0, 0)),
            pl.BlockSpec((bblk, s_len, d), lambda i: (i, 0, 0)),
        ],
        out_specs=pl.BlockSpec((bblk, s_len, d), lambda i: (i, 0, 0)),
        out_shape=jax.ShapeDtypeStruct((b, s_len, d), x.dtype),
        compiler_params=pltpu.CompilerParams(
            dimension_semantics=("arbitrary",)),
    )(pos_embed, kf_table, ln_weight.reshape(1, d), ln_bias.reshape(1, d), x)


# speedup vs baseline: 1.0231x; 1.0018x over previous
import functools

import jax
import jax.numpy as jnp
from jax import lax
from jax.experimental import pallas as pl
from jax.experimental.pallas import tpu as pltpu
from jax.experimental.pallas import tpu_sc as plsc

def _sc_gather_kernel(kf_hbm, idx_hbm, out_hbm, idx16, rows16, idx8, rows8,
                      sem, *, n_full, tail_base):
    wid = lax.axis_index("s") + lax.axis_index("c")

    @pl.when(wid < n_full)
    def _():
        base = wid * 16
        pltpu.sync_copy(idx_hbm.at[pl.ds(base, 16)], idx16)
        pltpu.async_copy(kf_hbm.at[idx16], rows16, sem).wait()
        pltpu.sync_copy(rows16, out_hbm.at[pl.ds(base, 16)])

    @pl.when(wid == n_full)
    def _():
        pltpu.sync_copy(idx_hbm.at[pl.ds(tail_base, 8)], idx8)
        pltpu.async_copy(kf_hbm.at[idx8], rows8, sem).wait()
        pltpu.sync_copy(rows8, out_hbm.at[pl.ds(tail_base, 8)])


def _gather_kf_sc(kf_embed, kf_idx, d):
    s_len = kf_idx.shape[0]
    n_full = s_len // 16
    mesh = plsc.VectorSubcoreMesh(core_axis_name="c", subcore_axis_name="s",
                                  num_cores=1)
    return pl.kernel(
        functools.partial(_sc_gather_kernel, n_full=n_full,
                          tail_base=n_full * 16),
        out_type=jax.ShapeDtypeStruct((s_len, d), jnp.float32),
        mesh=mesh,
        scratch_types=[
            pltpu.VMEM((16,), jnp.int32),
            pltpu.VMEM((16, d), jnp.float32),
            pltpu.VMEM((8,), jnp.int32),
            pltpu.VMEM((8, d), jnp.float32),
            pltpu.SemaphoreType.DMA,
        ],
    )(kf_embed, kf_idx)


def _tc_ln_kernel(pos_ref, kf_ref, w_ref, b_ref, x_ref, o_ref):
    bblk, s_len, d = x_ref.shape
    jmat = jnp.full((d, d), 1.0 / d, dtype=jnp.float32)
    add = (pos_ref[...] + kf_ref[...])[None, :, :]
    emb = (x_ref[...] + add).reshape(bblk * s_len, d)
    mean = lax.dot(emb, jmat,
                   preferred_element_type=jnp.float32)
    diff = emb - mean
    s2 = jnp.sum(diff * diff, axis=-1, keepdims=True)
    w = w_ref[...] * jnp.float32(d ** 0.5)
    scale = lax.rsqrt(s2 + jnp.float32(d * 1e-5))
    o_ref[...] = (diff * (scale * w) + b_ref[...]).reshape(bblk, s_len, d)


def kernel(x, pos_embed, kf_embed, ln_weight, ln_bias, n_past, n_future,
           n_trans):
    b, s_len, d = x.shape
    bblk = 64

    pos_index = jnp.arange(s_len, dtype=jnp.int32)
    n_past = jnp.asarray(n_past, jnp.int32)
    n_trans = jnp.asarray(n_trans, jnp.int32)
    n_position = n_past + jnp.asarray(n_future, jnp.int32) + n_trans
    kf_idx = jnp.where((pos_index >= n_past) & (pos_index < n_past + n_trans),
                       jnp.int32(1), jnp.int32(0))
    kf_idx = jnp.where(pos_index >= n_position, jnp.int32(2), kf_idx)

    kf_table = _gather_kf_sc(kf_embed, kf_idx, d)

    return pl.pallas_call(
        _tc_ln_kernel,
        grid=(b // bblk,),
        in_specs=[
            pl.BlockSpec((s_len, d), lambda i: (0, 0)),
            pl.BlockSpec((s_len, d), lambda i: (0, 0)),
            pl.BlockSpec((1, d), lambda i: (0, 0)),
            pl.BlockSpec((1, d), lambda i: (0, 0)),
            pl.BlockSpec((bblk, s_len, d), lambda i: (i, 0, 0)),
        ],
        out_specs=pl.BlockSpec((bblk, s_len, d), lambda i: (i, 0, 0)),
        out_shape=jax.ShapeDtypeStruct((b, s_len, d), x.dtype),
        compiler_params=pltpu.CompilerParams(
            dimension_semantics=("arbitrary",)),
    )(pos_embed, kf_table, ln_weight.reshape(1, d), ln_bias.reshape(1, d), x)
